# baseline (speedup 1.0000x reference)
import functools

import jax
import jax.numpy as jnp
from jax.experimental import pallas as pl
from jax.experimental.pallas import tpu as pltpu

N_EMBED = 1024
NUM_EXPERTS = 8
TOP_K = 2


def _router_kernel(x_ref, wg_ref, bg_ref, wn_ref, bn_ref, eps_ref, probs_ref):
    xt = x_ref[...]
    logits = jnp.dot(xt, wg_ref[...], preferred_element_type=jnp.float32) + bg_ref[...]
    noise = jax.nn.softplus(
        jnp.dot(xt, wn_ref[...], preferred_element_type=jnp.float32) + bn_ref[...])
    nl = logits + eps_ref[...] * noise
    lane = jax.lax.broadcasted_iota(jnp.int32, nl.shape, 1)
    v1 = jnp.max(nl, axis=-1, keepdims=True)
    i1 = jnp.min(jnp.where(nl == v1, lane, NUM_EXPERTS), axis=-1, keepdims=True)
    nl2 = jnp.where(lane == i1, -jnp.inf, nl)
    v2 = jnp.max(nl2, axis=-1, keepdims=True)
    i2 = jnp.min(jnp.where(nl2 == v2, lane, NUM_EXPERTS), axis=-1, keepdims=True)
    e2 = jnp.exp(v2 - v1)
    denom = 1.0 + e2
    probs_ref[...] = jnp.where(
        lane == i1, 1.0 / denom, jnp.where(lane == i2, e2 / denom, 0.0))


def _ffn_kernel(x_ref, w1_ref, b1_ref, w2_ref, b2_ref, p_ref, out_ref):
    e = pl.program_id(1)
    j = pl.program_id(2)

    @pl.when((e == 0) & (j == 0))
    def _():
        out_ref[...] = jnp.zeros_like(out_ref)

    xt = x_ref[...]
    h = jnp.maximum(
        jnp.dot(xt, w1_ref[0], preferred_element_type=jnp.float32) + b1_ref[0], 0.0)
    y = jnp.dot(h, w2_ref[0], preferred_element_type=jnp.float32)
    lane = jax.lax.broadcasted_iota(jnp.int32, p_ref.shape, 1)
    pe = jnp.sum(jnp.where(lane == e, p_ref[...], 0.0), axis=1, keepdims=True)

    @pl.when(j == 0)
    def _():
        out_ref[...] += pe * b2_ref[0]

    out_ref[...] += pe * y


def kernel(x, Wg, bg, Wn, bn, W1, b1, W2, b2):
    B, S, D = x.shape
    E = NUM_EXPERTS
    H = W1.shape[-1]
    N = B * S
    xf = x.reshape(N, D)
    eps = jax.random.normal(
        jax.random.key(42), (B, S, E), dtype=jnp.float32).reshape(N, E)

    RT = 2048
    probs = pl.pallas_call(
        _router_kernel,
        grid=(N // RT,),
        in_specs=[
            pl.BlockSpec((RT, D), lambda i: (i, 0)),
            pl.BlockSpec((D, E), lambda i: (0, 0)),
            pl.BlockSpec((1, E), lambda i: (0, 0)),
            pl.BlockSpec((D, E), lambda i: (0, 0)),
            pl.BlockSpec((1, E), lambda i: (0, 0)),
            pl.BlockSpec((RT, E), lambda i: (i, 0)),
        ],
        out_specs=pl.BlockSpec((RT, E), lambda i: (i, 0)),
        out_shape=jax.ShapeDtypeStruct((N, E), jnp.float32),
    )(xf, Wg, bg.reshape(1, E), Wn, bn.reshape(1, E), eps)

    T = 2048
    HT = 512
    out = pl.pallas_call(
        _ffn_kernel,
        grid=(N // T, E, H // HT),
        in_specs=[
            pl.BlockSpec((T, D), lambda i, e, j: (i, 0)),
            pl.BlockSpec((1, D, HT), lambda i, e, j: (e, 0, j)),
            pl.BlockSpec((1, 1, HT), lambda i, e, j: (e, 0, j)),
            pl.BlockSpec((1, HT, D), lambda i, e, j: (e, j, 0)),
            pl.BlockSpec((1, 1, D), lambda i, e, j: (e, 0, 0)),
            pl.BlockSpec((T, E), lambda i, e, j: (i, 0)),
        ],
        out_specs=pl.BlockSpec((T, D), lambda i, e, j: (i, 0)),
        out_shape=jax.ShapeDtypeStruct((N, D), jnp.float32),
    )(xf, W1, b1.reshape(E, 1, H), W2, b2.reshape(E, 1, D), probs)
    return out.reshape(B, S, D)


# speedup vs baseline: 2.1378x; 2.1378x over previous
import functools

import jax
import jax.numpy as jnp
from jax import lax
from jax.experimental import pallas as pl
from jax.experimental.pallas import tpu as pltpu
from jax.experimental.pallas import tpu_sc as plsc

D = 1024
E = 8
H = 4096
N = 8192
TILE = 256
CAP = N * 2 + E * TILE
NTILES = CAP // TILE

NC = 2
NS = 16
NW = NC * NS
CHUNK = N // NW
SUB = 64
NSUB = CHUNK // SUB


def _topk_kernel(x_ref, wg_ref, bg_ref, wn_ref, bn_ref, eps_ref,
                 p1_ref, p2_ref, oh1_ref, oh2_ref):
    xt = x_ref[...]
    logits = jnp.dot(xt, wg_ref[...], preferred_element_type=jnp.float32) + bg_ref[...]
    noise = jax.nn.softplus(
        jnp.dot(xt, wn_ref[...], preferred_element_type=jnp.float32) + bn_ref[...])
    nl = logits + eps_ref[...] * noise
    lane = jax.lax.broadcasted_iota(jnp.int32, nl.shape, 1)
    v1 = jnp.max(nl, axis=-1, keepdims=True)
    i1 = jnp.min(jnp.where(nl == v1, lane, E), axis=-1, keepdims=True)
    nl2 = jnp.where(lane == i1, -jnp.inf, nl)
    v2 = jnp.max(nl2, axis=-1, keepdims=True)
    i2 = jnp.min(jnp.where(nl2 == v2, lane, E), axis=-1, keepdims=True)
    e2 = jnp.exp(v2 - v1)
    denom = 1.0 + e2
    p1_ref[...] = 1.0 / denom
    p2_ref[...] = e2 / denom
    oh1_ref[...] = (lane == i1).astype(jnp.float32)
    oh2_ref[...] = (lane == i2).astype(jnp.float32)


_G = 128
_NG = N // _G


def _dispatch_pos_kernel(oh1_ref, oh2_ref, pos1_ref, pos2_ref, texp_ref):
    counts1 = jnp.sum(oh1_ref[...], axis=0, keepdims=True)
    counts = counts1 + jnp.sum(oh2_ref[...], axis=0, keepdims=True)
    padded = jnp.ceil(counts * (1.0 / TILE)) * float(TILE)
    upper = (jax.lax.broadcasted_iota(jnp.int32, (E, E), 0)
             < jax.lax.broadcasted_iota(jnp.int32, (E, E), 1)).astype(jnp.float32)
    start = jnp.dot(padded, upper, preferred_element_type=jnp.float32)
    ltri = (jax.lax.broadcasted_iota(jnp.int32, (_G, _G), 0)
            >= jax.lax.broadcasted_iota(jnp.int32, (_G, _G), 1)).astype(jnp.float32)
    base1 = start
    base2 = start + counts1

    def body(g, run):
        run1, run2 = run
        sl = pl.ds(g * _G, _G)
        oh1 = oh1_ref[sl, :]
        oh2 = oh2_ref[sl, :]
        inc1 = jnp.dot(ltri, oh1, preferred_element_type=jnp.float32)
        inc2 = jnp.dot(ltri, oh2, preferred_element_type=jnp.float32)
        pos1_ref[sl, :] = jnp.sum(
            oh1 * (base1 + run1 + inc1 - oh1), axis=1,
            keepdims=True).astype(jnp.int32)
        pos2_ref[sl, :] = jnp.sum(
            oh2 * (base2 + run2 + inc2 - oh2), axis=1,
            keepdims=True).astype(jnp.int32)
        return (run1 + inc1[_G - 1:_G, :], run2 + inc2[_G - 1:_G, :])

    zero = jnp.zeros((1, E), jnp.float32)
    lax.fori_loop(0, _NG, body, (zero, zero))

    row = (jax.lax.broadcasted_iota(jnp.int32, (NTILES, E), 0)
           .astype(jnp.float32) * float(TILE))
    texp_ref[...] = (jnp.sum((row >= start).astype(jnp.int32), axis=1,
                             keepdims=True) - 1)


def _d1_kernel(te_ref, xs_ref, w1_ref, b1_ref, h_ref):
    del te_ref
    h_ref[...] = jnp.maximum(
        jnp.dot(xs_ref[...], w1_ref[0], preferred_element_type=jnp.float32)
        + b1_ref[0], 0.0)


def _d2_kernel(te_ref, h_ref, w2_ref, b2_ref, ys_ref):
    del te_ref
    ys_ref[...] = (
        jnp.dot(h_ref[...], w2_ref[0], preferred_element_type=jnp.float32)
        + b2_ref[0])


def _combine_kernel(p1_ref, p2_ref, y1_ref, y2_ref, out_ref):
    out_ref[...] = p1_ref[...] * y1_ref[...] + p2_ref[...] * y2_ref[...]


def _sc_mesh():
    return plsc.VectorSubcoreMesh(core_axis_name="c", subcore_axis_name="s")


def _dispatch_sc(xf, pos1, pos2):
    @functools.partial(
        pl.kernel,
        mesh=_sc_mesh(),
        out_type=jax.ShapeDtypeStruct((CAP, D), jnp.float32),
        scratch_types=[
            pltpu.VMEM((SUB, D), jnp.float32),
            pltpu.VMEM((SUB,), jnp.int32),
            pltpu.VMEM((SUB,), jnp.int32),
            pltpu.SemaphoreType.DMA,
        ],
    )
    def k(xf_hbm, pos1_hbm, pos2_hbm, xs_hbm, rows_v, idx1_v, idx2_v, sem):
        wid = lax.axis_index("s") * NC + lax.axis_index("c")
        base0 = wid * CHUNK

        def body(it, carry):
            base = base0 + it * SUB
            pltpu.sync_copy(xf_hbm.at[pl.ds(base, SUB)], rows_v)
            pltpu.sync_copy(pos1_hbm.at[pl.ds(base, SUB)], idx1_v)
            pltpu.sync_copy(pos2_hbm.at[pl.ds(base, SUB)], idx2_v)
            pltpu.async_copy(rows_v, xs_hbm.at[idx1_v], sem).wait()
            pltpu.async_copy(rows_v, xs_hbm.at[idx2_v], sem).wait()
            return carry

        lax.fori_loop(0, NSUB, body, 0)

    return k(xf, pos1, pos2)


def _collect_sc(ys, pos1, pos2):
    @functools.partial(
        pl.kernel,
        mesh=_sc_mesh(),
        out_type=(jax.ShapeDtypeStruct((N, D), jnp.float32),
                  jax.ShapeDtypeStruct((N, D), jnp.float32)),
        scratch_types=[
            pltpu.VMEM((SUB, D), jnp.float32),
            pltpu.VMEM((SUB,), jnp.int32),
            pltpu.SemaphoreType.DMA,
        ],
    )
    def k(ys_hbm, pos1_hbm, pos2_hbm, y1_hbm, y2_hbm, buf_v, idx_v, sem):
        wid = lax.axis_index("s") * NC + lax.axis_index("c")
        base0 = wid * CHUNK

        def body(it, carry):
            base = base0 + it * SUB
            pltpu.sync_copy(pos1_hbm.at[pl.ds(base, SUB)], idx_v)
            pltpu.async_copy(ys_hbm.at[idx_v], buf_v, sem).wait()
            pltpu.sync_copy(buf_v, y1_hbm.at[pl.ds(base, SUB)])
            pltpu.sync_copy(pos2_hbm.at[pl.ds(base, SUB)], idx_v)
            pltpu.async_copy(ys_hbm.at[idx_v], buf_v, sem).wait()
            pltpu.sync_copy(buf_v, y2_hbm.at[pl.ds(base, SUB)])
            return carry

        lax.fori_loop(0, NSUB, body, 0)

    return k(ys, pos1, pos2)


def kernel(x, Wg, bg, Wn, bn, W1, b1, W2, b2):
    B, S, _ = x.shape
    xf = x.reshape(N, D)
    eps = jax.random.normal(
        jax.random.key(42), (B, S, E), dtype=jnp.float32).reshape(N, E)

    RT = 1024
    p1, p2, oh1, oh2 = pl.pallas_call(
        _topk_kernel,
        grid=(N // RT,),
        in_specs=[
            pl.BlockSpec((RT, D), lambda i: (i, 0)),
            pl.BlockSpec((D, E), lambda i: (0, 0)),
            pl.BlockSpec((1, E), lambda i: (0, 0)),
            pl.BlockSpec((D, E), lambda i: (0, 0)),
            pl.BlockSpec((1, E), lambda i: (0, 0)),
            pl.BlockSpec((RT, E), lambda i: (i, 0)),
        ],
        out_specs=[
            pl.BlockSpec((RT, 1), lambda i: (i, 0)),
            pl.BlockSpec((RT, 1), lambda i: (i, 0)),
            pl.BlockSpec((RT, E), lambda i: (i, 0)),
            pl.BlockSpec((RT, E), lambda i: (i, 0)),
        ],
        out_shape=[
            jax.ShapeDtypeStruct((N, 1), jnp.float32),
            jax.ShapeDtypeStruct((N, 1), jnp.float32),
            jax.ShapeDtypeStruct((N, E), jnp.float32),
            jax.ShapeDtypeStruct((N, E), jnp.float32),
        ],
    )(xf, Wg, bg.reshape(1, E), Wn, bn.reshape(1, E), eps)

    pos1, pos2, texp = pl.pallas_call(
        _dispatch_pos_kernel,
        out_shape=[
            jax.ShapeDtypeStruct((N, 1), jnp.int32),
            jax.ShapeDtypeStruct((N, 1), jnp.int32),
            jax.ShapeDtypeStruct((NTILES, 1), jnp.int32),
        ],
    )(oh1, oh2)

    pos1f = pos1.reshape(N)
    pos2f = pos2.reshape(N)
    xs = _dispatch_sc(xf, pos1f, pos2f)

    texp_flat = texp.reshape(NTILES)
    h = pl.pallas_call(
        _d1_kernel,
        grid_spec=pltpu.PrefetchScalarGridSpec(
            num_scalar_prefetch=1,
            grid=(NTILES,),
            in_specs=[
                pl.BlockSpec((TILE, D), lambda i, te: (i, 0)),
                pl.BlockSpec((1, D, H), lambda i, te: (te[i], 0, 0)),
                pl.BlockSpec((1, 1, H), lambda i, te: (te[i], 0, 0)),
            ],
            out_specs=pl.BlockSpec((TILE, H), lambda i, te: (i, 0)),
        ),
        out_shape=jax.ShapeDtypeStruct((CAP, H), jnp.float32),
    )(texp_flat, xs, W1, b1.reshape(E, 1, H))

    ys = pl.pallas_call(
        _d2_kernel,
        grid_spec=pltpu.PrefetchScalarGridSpec(
            num_scalar_prefetch=1,
            grid=(NTILES,),
            in_specs=[
                pl.BlockSpec((TILE, H), lambda i, te: (i, 0)),
                pl.BlockSpec((1, H, D), lambda i, te: (te[i], 0, 0)),
                pl.BlockSpec((1, 1, D), lambda i, te: (te[i], 0, 0)),
            ],
            out_specs=pl.BlockSpec((TILE, D), lambda i, te: (i, 0)),
        ),
        out_shape=jax.ShapeDtypeStruct((CAP, D), jnp.float32),
    )(texp_flat, h, W2, b2.reshape(E, 1, D))

    y1, y2 = _collect_sc(ys, pos1f, pos2f)

    CT = 2048
    out = pl.pallas_call(
        _combine_kernel,
        grid=(N // CT,),
        in_specs=[
            pl.BlockSpec((CT, 1), lambda i: (i, 0)),
            pl.BlockSpec((CT, 1), lambda i: (i, 0)),
            pl.BlockSpec((CT, D), lambda i: (i, 0)),
            pl.BlockSpec((CT, D), lambda i: (i, 0)),
        ],
        out_specs=pl.BlockSpec((CT, D), lambda i: (i, 0)),
        out_shape=jax.ShapeDtypeStruct((N, D), jnp.float32),
    )(p1, p2, y1, y2)
    return out.reshape(B, S, D)
